# baseline (speedup 1.0000x reference)
import jax
import jax.numpy as jnp
from jax.experimental import pallas as pl

_DEPTH = 3
_MAX_NB = 10


def _body(ia_ref, ib_ref, idxa_ref, idxb_ref, nn_ref, nm_ref,
          W_emb_ref, W_na_ref, W_nb_ref, W_self_ref,
          W_U2a_ref, W_U2b_ref, b_U2_ref, W_U1a_ref, W_U1b_ref, b_U1_ref,
          outk_ref, outm_ref):
    K = _MAX_NB
    N = ia_ref.shape[1]
    NB = ib_ref.shape[1]
    H = W_emb_ref.shape[1]
    f32 = jnp.float32

    def dot(a, b):
        return jax.lax.dot_general(a, b, (((1,), (0,)), ((), ())),
                                   preferred_element_type=f32)

    def dot16(a, b):
        return jax.lax.dot_general(a.astype(jnp.bfloat16),
                                   b.astype(jnp.bfloat16),
                                   (((1,), (0,)), ((), ())),
                                   preferred_element_type=f32)

    ia = ia_ref[0]
    ib = ib_ref[0]
    idxa = idxa_ref[0]
    idxb = idxb_ref[0]
    nn = nn_ref[0]
    nm = nm_ref[0]

    af = jax.nn.relu(dot(ia, W_emb_ref[...]))

    kio = jax.lax.broadcasted_iota(jnp.int32, (K, N), 0)
    mask = (kio < nn).astype(f32)[:, :, None]

    iot_a = jax.lax.broadcasted_iota(jnp.int32, (K, N, N), 2)
    oha = (idxa[:, :, None] == iot_a).astype(f32).reshape(K * N, N)
    iot_b = jax.lax.broadcasted_iota(jnp.int32, (K, N, NB), 2)
    ohb = (idxb[:, :, None] == iot_b).astype(f32).reshape(K * N, NB)

    fbg = dot16(ohb, ib)
    b1g = dot(fbg, W_nb_ref[...]).reshape(K, N, H)
    b2g = dot(fbg, W_U2b_ref[...]).reshape(K, N, H)
    b_u2 = b_U2_ref[...][None]
    b_u1 = b_U1_ref[...]

    for _ in range(_DEPTH - 1):
        a2 = dot(af, W_U2a_ref[...])
        g2 = dot16(oha, a2).reshape(K, N, H)
        nl = jnp.sum(jax.nn.relu(g2 + b2g + b_u2) * mask, axis=0)
        af = jax.nn.relu(dot(af, W_U1a_ref[...]) + dot(nl, W_U1b_ref[...])
                         + b_u1)

    a1 = dot(af, W_na_ref[...])
    g1 = dot16(oha, a1).reshape(K, N, H)
    f_nei = jnp.sum(g1 * b1g * mask, axis=0)
    f_self = dot(af, W_self_ref[...])
    out = f_nei * f_self * nm
    outk_ref[0] = out
    outm_ref[0] = jnp.sum(out, axis=0, keepdims=True)


@jax.jit
def kernel(input_atom, input_bond, atom_graph, bond_graph, num_nbs,
           node_mask, W_emb, W_nei_atom, W_nei_bond, W_self, W_U2, b_U2,
           W_U1, b_U1):
    B, N, AF = input_atom.shape
    NB, BF = input_bond.shape[1], input_bond.shape[2]
    H = W_emb.shape[1]
    K = _MAX_NB

    idxa = jnp.transpose(atom_graph[..., 1], (0, 2, 1)).astype(jnp.int32)
    idxb = jnp.transpose(bond_graph[..., 1], (0, 2, 1)).astype(jnp.int32)
    nn3 = num_nbs.reshape(B, 1, N).astype(jnp.int32)
    W_U2a, W_U2b = W_U2[:H], W_U2[H:]
    W_U1a, W_U1b = W_U1[:H], W_U1[H:]
    b_U2r = b_U2.reshape(1, H)
    b_U1r = b_U1.reshape(1, H)

    full = lambda *shape: pl.BlockSpec(shape, lambda b: (0,) * len(shape))
    per_mol = lambda *shape: pl.BlockSpec((1,) + shape,
                                          lambda b: (b,) + (0,) * len(shape))

    outk, outm = pl.pallas_call(
        _body,
        grid=(B,),
        in_specs=[
            per_mol(N, AF),
            per_mol(NB, BF),
            per_mol(K, N),
            per_mol(K, N),
            per_mol(1, N),
            per_mol(N, 1),
            full(AF, H),
            full(H, H),
            full(BF, H),
            full(H, H),
            full(H, H),
            full(BF, H),
            full(1, H),
            full(H, H),
            full(H, H),
            full(1, H),
        ],
        out_specs=[per_mol(N, H), per_mol(1, H)],
        out_shape=[
            jax.ShapeDtypeStruct((B, N, H), jnp.float32),
            jax.ShapeDtypeStruct((B, 1, H), jnp.float32),
        ],
    )(input_atom, input_bond, idxa, idxb, nn3, node_mask,
      W_emb, W_nei_atom, W_nei_bond, W_self,
      W_U2a, W_U2b, b_U2r, W_U1a, W_U1b, b_U1r)
    return outk, outm.reshape(B, H)


# speedup vs baseline: 1.0252x; 1.0252x over previous
import jax
import jax.numpy as jnp
from jax.experimental import pallas as pl

_DEPTH = 3
_MAX_NB = 10


def _body(ia_ref, ib_ref, idxa_ref, idxb_ref, nn_ref, nm_ref,
          W_emb_ref, W_nas_ref, W_nb_ref,
          W_U2a_ref, W_U2b_ref, b_U2_ref, W_U1a_ref, W_U1b_ref, b_U1_ref,
          outk_ref, outm_ref):
    K = _MAX_NB
    N = ia_ref.shape[1]
    NB = ib_ref.shape[1]
    H = W_emb_ref.shape[1]
    f32 = jnp.float32

    def dot(a, b):
        return jax.lax.dot_general(a, b, (((1,), (0,)), ((), ())),
                                   preferred_element_type=f32)

    ia = ia_ref[0]
    ib = ib_ref[0]
    idxa = idxa_ref[0]
    idxb = idxb_ref[0]
    nn = nn_ref[0]
    nm = nm_ref[0]

    af = jax.nn.relu(dot(ia, W_emb_ref[...]))

    kio = jax.lax.broadcasted_iota(jnp.int32, (K, N), 0)
    mask = (kio < nn).astype(f32)[:, :, None]

    bf16 = jnp.bfloat16
    iot_a = jax.lax.broadcasted_iota(jnp.int32, (K, N, N), 2)
    oham = ((idxa[:, :, None] == iot_a).astype(bf16)
            * mask.astype(bf16)).reshape(K * N, N)
    iot_b = jax.lax.broadcasted_iota(jnp.int32, (K, N, NB), 2)
    ohb = (idxb[:, :, None] == iot_b).astype(bf16).reshape(K * N, NB)

    fbg = dot(ohb, ib.astype(bf16))
    b1g = dot(fbg, W_nb_ref[...]).reshape(K, N, H)
    b_u2 = b_U2_ref[...][None]
    b_u1 = b_U1_ref[...]
    b2gm = (dot(fbg, W_U2b_ref[...]).reshape(K, N, H) + b_u2) * mask

    for _ in range(_DEPTH - 1):
        a2 = dot(af, W_U2a_ref[...])
        g2m = dot(oham, a2.astype(bf16)).reshape(K, N, H)
        nl = jnp.sum(jax.nn.relu(g2m + b2gm), axis=0)
        af = jax.nn.relu(dot(af, W_U1a_ref[...]) + dot(nl, W_U1b_ref[...])
                         + b_u1)

    fs_a1 = dot(af, W_nas_ref[...])
    g1m = dot(oham, fs_a1[:, :H].astype(bf16)).reshape(K, N, H)
    f_nei = jnp.sum(g1m * b1g, axis=0)
    out = f_nei * fs_a1[:, H:] * nm
    outk_ref[0] = out
    outm_ref[0] = jnp.sum(out, axis=0, keepdims=True)


@jax.jit
def kernel(input_atom, input_bond, atom_graph, bond_graph, num_nbs,
           node_mask, W_emb, W_nei_atom, W_nei_bond, W_self, W_U2, b_U2,
           W_U1, b_U1):
    B, N, AF = input_atom.shape
    NB, BF = input_bond.shape[1], input_bond.shape[2]
    H = W_emb.shape[1]
    K = _MAX_NB

    idxa = jnp.transpose(atom_graph[..., 1], (0, 2, 1)).astype(jnp.int32)
    idxb = jnp.transpose(bond_graph[..., 1], (0, 2, 1)).astype(jnp.int32)
    nn3 = num_nbs.reshape(B, 1, N).astype(jnp.int32)
    W_U2a, W_U2b = W_U2[:H], W_U2[H:]
    W_U1a, W_U1b = W_U1[:H], W_U1[H:]
    b_U2r = b_U2.reshape(1, H)
    b_U1r = b_U1.reshape(1, H)

    full = lambda *shape: pl.BlockSpec(shape, lambda b: (0,) * len(shape))
    per_mol = lambda *shape: pl.BlockSpec((1,) + shape,
                                          lambda b: (b,) + (0,) * len(shape))

    outk, outm = pl.pallas_call(
        _body,
        grid=(B,),
        in_specs=[
            per_mol(N, AF),
            per_mol(NB, BF),
            per_mol(K, N),
            per_mol(K, N),
            per_mol(1, N),
            per_mol(N, 1),
            full(AF, H),
            full(H, 2 * H),
            full(BF, H),
            full(H, H),
            full(BF, H),
            full(1, H),
            full(H, H),
            full(H, H),
            full(1, H),
        ],
        out_specs=[per_mol(N, H), per_mol(1, H)],
        out_shape=[
            jax.ShapeDtypeStruct((B, N, H), jnp.float32),
            jax.ShapeDtypeStruct((B, 1, H), jnp.float32),
        ],
    )(input_atom, input_bond, idxa, idxb, nn3, node_mask,
      W_emb, jnp.concatenate([W_nei_atom, W_self], axis=1), W_nei_bond,
      W_U2a, W_U2b, b_U2r, W_U1a, W_U1b, b_U1r)
    return outk, outm.reshape(B, H)
